# baseline (speedup 1.0000x reference)
import functools

import jax
import jax.numpy as jnp
from jax import lax
from jax.experimental import pallas as pl
from jax.experimental.pallas import tpu as pltpu
from jax.experimental.pallas import tpu_sc as plsc

B, L = 16384, 64
NC, NS = 2, 16
NW = NC * NS
ROWS = B // NW
LANES = 16
KBITS = ROWS // LANES

_mesh = plsc.VectorSubcoreMesh(
    core_axis_name="c", subcore_axis_name="s", num_cores=NC, num_subcores=NS
)


@functools.partial(
    pl.kernel,
    out_type=(
        jax.ShapeDtypeStruct((B,), jnp.float32),
        jax.ShapeDtypeStruct((B,), jnp.int32),
    ),
    mesh=_mesh,
    scratch_types=(
        pltpu.VMEM((L, ROWS), jnp.int32),
        pltpu.VMEM((10, LANES), jnp.float32),
        pltpu.VMEM((ROWS,), jnp.float32),
        pltpu.VMEM((ROWS,), jnp.int32),
    ),
    compiler_params=pltpu.CompilerParams(needs_layout_passes=False),
)
def _sc_walk(bits_hbm, pw_hbm, pred_hbm, sym_hbm,
             bits_v, pw_v, pred_v, sym_v):
    wid = lax.axis_index("s") * NC + lax.axis_index("c")
    base = wid * ROWS
    pltpu.sync_copy(pw_hbm, pw_v)
    pltpu.sync_copy(bits_hbm.at[:, pl.ds(base, ROWS)], bits_v)

    big = jnp.full((LANES,), 3.4e38, jnp.float32)
    one = jnp.full((LANES,), 1, jnp.int32)
    zero = jnp.zeros((LANES,), jnp.int32)
    ones = jnp.full((LANES,), -1, jnp.int32)
    fone = jnp.full((LANES,), 1.0, jnp.float32)
    lane = lax.iota(jnp.int32, LANES)

    def pair(k):
        y0 = pw_v[2 * k]
        y1 = pw_v[2 * k + 1]
        t = fone / (fone + jnp.exp(-jnp.abs(y0 - y1)))
        m = jnp.where(y1 > y0, ones, zero)
        return t, m

    ftv, s0m = pair(0)
    cells = [pair(k + 1) for k in range(4)]
    tv = [c[0] for c in cells]
    m00, m01, m10, m11 = (c[1] for c in cells)

    def step(i, carry):
        st, v00, v01, v10, v11 = carry
        col = jnp.full((LANES,), i, jnp.int32)
        vals = [bits_v[i, pl.ds(16 * k, LANES)] << k for k in range(KBITS)]
        while len(vals) > 1:
            vals = [vals[j] | vals[j + 1] for j in range(0, len(vals) - 1, 2)] + (
                [vals[-1]] if len(vals) % 2 else []
            )
        a = vals[0]
        na = ~a
        ns = ~st
        nst = (na & ((ns & m00) | (st & m01))) | (a & ((ns & m10) | (st & m11)))
        nns = ~nst
        return (nst,
                v00 | (na & nns), v01 | (na & nst),
                v10 | (a & nns), v11 | (a & nst))

    init = (s0m, zero, zero, zero, zero)
    st, v00, v01, v10, v11 = lax.fori_loop(0, L, step, init)

    vs = (v00, v01, v10, v11)

    def decode(k, carry):
        pred = ftv
        for j in range(4):
            hit = ((vs[j] >> k) & one) == one
            pred = jnp.minimum(pred, jnp.where(hit, tv[j], big))
        pred_v[pl.ds(16 * k, LANES)] = pred
        sym_v[pl.ds(16 * k, LANES)] = (st >> k) & one
        return carry

    lax.fori_loop(0, KBITS, decode, 0)

    pltpu.sync_copy(pred_v, pred_hbm.at[pl.ds(base, ROWS)])
    pltpu.sync_copy(sym_v, sym_hbm.at[pl.ds(base, ROWS)])


def kernel(binary_list, eval, weight_initial, weights):
    pw = jnp.concatenate(
        [weight_initial.reshape(-1), weights.reshape(-1)]
    )[:, None] + jnp.zeros((1, LANES), jnp.float32)
    pred, sym = _sc_walk(binary_list.T, pw)
    return pred, sym.reshape(B, 1, 1)


# speedup vs baseline: 1.0125x; 1.0125x over previous
import functools

import jax
import jax.numpy as jnp
from jax import lax
from jax.experimental import pallas as pl
from jax.experimental.pallas import tpu as pltpu
from jax.experimental.pallas import tpu_sc as plsc

B, L = 16384, 64
NC, NS = 2, 16
NW = NC * NS
ROWS = B // NW
LANES = 16
KBITS = ROWS // LANES

_mesh = plsc.VectorSubcoreMesh(
    core_axis_name="c", subcore_axis_name="s", num_cores=NC, num_subcores=NS
)


@functools.partial(
    pl.kernel,
    out_type=(
        jax.ShapeDtypeStruct((B,), jnp.float32),
        jax.ShapeDtypeStruct((B,), jnp.int32),
    ),
    mesh=_mesh,
    scratch_types=(
        pltpu.VMEM((L, ROWS), jnp.int32),
        pltpu.VMEM((10, LANES), jnp.float32),
        pltpu.VMEM((ROWS,), jnp.float32),
        pltpu.VMEM((ROWS,), jnp.int32),
        pltpu.SemaphoreType.DMA,
        pltpu.SemaphoreType.DMA,
        pltpu.SemaphoreType.DMA,
        pltpu.SemaphoreType.DMA,
    ),
    compiler_params=pltpu.CompilerParams(needs_layout_passes=False),
)
def _sc_walk(bits_hbm, pw_hbm, pred_hbm, sym_hbm,
             bits_v, pw_v, pred_v, sym_v, sem0, sem1, sem2, sem3):
    wid = lax.axis_index("s") * NC + lax.axis_index("c")
    base = wid * ROWS
    sems = (sem0, sem1, sem2, sem3)
    nchunk = len(sems)
    csteps = L // nchunk
    copies = [
        pltpu.async_copy(
            bits_hbm.at[pl.ds(c * csteps, csteps), pl.ds(base, ROWS)],
            bits_v.at[pl.ds(c * csteps, csteps), :],
            sems[c],
        )
        for c in range(nchunk)
    ]
    pltpu.sync_copy(pw_hbm, pw_v)

    big = jnp.full((LANES,), 3.4e38, jnp.float32)
    one = jnp.full((LANES,), 1, jnp.int32)
    zero = jnp.zeros((LANES,), jnp.int32)
    ones = jnp.full((LANES,), -1, jnp.int32)
    fone = jnp.full((LANES,), 1.0, jnp.float32)
    lane = lax.iota(jnp.int32, LANES)

    def pair(k):
        y0 = pw_v[2 * k]
        y1 = pw_v[2 * k + 1]
        t = fone / (fone + jnp.exp(-jnp.abs(y0 - y1)))
        m = jnp.where(y1 > y0, ones, zero)
        return t, m

    ftv, s0m = pair(0)
    cells = [pair(k + 1) for k in range(4)]
    tv = [c[0] for c in cells]
    m00, m01, m10, m11 = (c[1] for c in cells)

    def step(i, carry):
        st, v00, v01, v10, v11 = carry
        col = jnp.full((LANES,), i, jnp.int32)
        vals = [bits_v[i, pl.ds(16 * k, LANES)] << k for k in range(KBITS)]
        while len(vals) > 1:
            vals = [vals[j] | vals[j + 1] for j in range(0, len(vals) - 1, 2)] + (
                [vals[-1]] if len(vals) % 2 else []
            )
        a = vals[0]
        na = ~a
        ns = ~st
        nst = (na & ((ns & m00) | (st & m01))) | (a & ((ns & m10) | (st & m11)))
        nns = ~nst
        return (nst,
                v00 | (na & nns), v01 | (na & nst),
                v10 | (a & nns), v11 | (a & nst))

    carry = (s0m, zero, zero, zero, zero)
    for c in range(nchunk):
        copies[c].wait()
        carry = lax.fori_loop(c * csteps, (c + 1) * csteps, step, carry)
    st, v00, v01, v10, v11 = carry

    vs = (v00, v01, v10, v11)

    def decode(k, carry):
        pred = ftv
        for j in range(4):
            hit = ((vs[j] >> k) & one) == one
            pred = jnp.minimum(pred, jnp.where(hit, tv[j], big))
        pred_v[pl.ds(16 * k, LANES)] = pred
        sym_v[pl.ds(16 * k, LANES)] = (st >> k) & one
        return carry

    lax.fori_loop(0, KBITS, decode, 0)

    pltpu.sync_copy(pred_v, pred_hbm.at[pl.ds(base, ROWS)])
    pltpu.sync_copy(sym_v, sym_hbm.at[pl.ds(base, ROWS)])


def kernel(binary_list, eval, weight_initial, weights):
    pw = jnp.concatenate(
        [weight_initial.reshape(-1), weights.reshape(-1)]
    )[:, None] + jnp.zeros((1, LANES), jnp.float32)
    pred, sym = _sc_walk(binary_list.T, pw)
    return pred, sym.reshape(B, 1, 1)
